# baseline (device time: 126640 ns/iter reference)
import sys

import jax
import jax.numpy as jnp
from jax import lax
from jax.experimental import pallas as pl
from jax.experimental.pallas import tpu as pltpu

N_DEV = 16
E_LOC = 4
E_HALF = 2

import os
_COMM_ONLY_PROBE = bool(os.environ.get("COMM_ONLY_PROBE"))


def _build_ring():
    try:
        import distributed_mesh_v7x as dm

        mesh = dm.get_mesh("i", N_DEV)
        coords = [tuple(d.coords) for d in mesh.devices.flat]
        xs = sorted({c[0] for c in coords})
        ys = sorted({c[1] for c in coords})
        zs = sorted({c[2] for c in coords})
        if (
            len(set(coords)) == N_DEV
            and len(xs) == 2
            and len(ys) == 2
            and len(zs) == 4
        ):
            path = [
                (0, 0, 0), (1, 0, 0), (1, 0, 1), (1, 0, 2),
                (1, 0, 3), (1, 1, 3), (1, 1, 2), (1, 1, 1),
                (1, 1, 0), (0, 1, 0), (0, 1, 1), (0, 1, 2),
                (0, 1, 3), (0, 0, 3), (0, 0, 2), (0, 0, 1),
            ]
            idx = {c: i for i, c in enumerate(coords)}
            ring = [idx[(xs[a], ys[b], zs[c])] for a, b, c in path]
            if sorted(ring) == list(range(N_DEV)):
                return ring
    except Exception as e:
        print(f"_build_ring fallback: {type(e).__name__}: {e}", file=sys.stderr)
    return list(range(N_DEV))


RING = _build_ring()
print(f"RING = {RING}", file=sys.stderr)
INV_RING = [0] * N_DEV
for _p, _l in enumerate(RING):
    INV_RING[_l] = _p


def kernel(x, router_W, route_idx, expert_W):
    T, D = x.shape
    E = router_W.shape[1]
    H = expert_W.shape[2]
    N_R = 8
    N_L = 7

    def body(x_ref, rw_ref, idx_ref, ew_ref, ring_ref, inv_ref, out_ref,
             ewb_ref, bufs, sends, recvs, creds):
        my = lax.axis_index("i")
        i16 = lax.broadcasted_iota(jnp.int32, (1, N_DEV), 1)
        ring_t = ring_ref[:, :]
        inv_t = inv_ref[:, :]

        def lut(tbl, i):
            return jnp.sum(jnp.where(i16 == i, tbl, 0))

        r = lut(inv_t, my)
        right = lut(ring_t, lax.rem(r + 1, N_DEV))
        left = lut(ring_t, lax.rem(r + N_DEV - 1, N_DEV))

        barrier_sem = pltpu.get_barrier_semaphore()
        for nbr in (left, right):
            pl.semaphore_signal(
                barrier_sem, inc=1,
                device_id=(nbr,), device_id_type=pl.DeviceIdType.MESH,
            )
        pl.semaphore_wait(barrier_sem, 2)

        ewb_ref[:, :, :] = ew_ref[:, :, :].astype(jnp.bfloat16)

        xv = x_ref[:, :]
        xb = xv.astype(jnp.bfloat16)
        scores = jnp.dot(xv, rw_ref[:, :], preferred_element_type=jnp.float32)
        e0 = idx_ref[:, 0:1]
        e1 = idx_ref[:, 1:2]
        cols = lax.broadcasted_iota(jnp.int32, (T, E), 1)
        s0 = jnp.sum(jnp.where(cols == e0, scores, 0.0), axis=1, keepdims=True)
        s1 = jnp.sum(jnp.where(cols == e1, scores, 0.0), axis=1, keepdims=True)
        w0 = 1.0 / (1.0 + jnp.exp(s1 - s0))
        w1 = 1.0 - w0

        def accum(origin, w_half, k_base, init):
            contrib = None
            for k in range(E_HALF):
                e_id = origin * E_LOC + k_base + k
                col = (jnp.where(e0 == e_id, w0, 0.0)
                       + jnp.where(e1 == e_id, w1, 0.0))
                xs = xb * col.astype(jnp.bfloat16)
                part = jnp.dot(xs, w_half[k],
                               preferred_element_type=jnp.float32)
                contrib = part if contrib is None else contrib + part
            if init:
                out_ref[:, :] = contrib
            else:
                out_ref[:, :] += contrib

        def make_rdma(s, dirn, half, h):
            if h == 0:
                src = ewb_ref.at[pl.ds(half * E_HALF, E_HALF)]
            else:
                src = bufs.at[s, h % 2]
            return pltpu.make_async_remote_copy(
                src_ref=src,
                dst_ref=bufs.at[s, (h + 1) % 2],
                send_sem=sends.at[s, h % 2],
                recv_sem=recvs.at[s, (h + 1) % 2],
                device_id=(right if dirn == "R" else left,),
                device_id_type=pl.DeviceIdType.MESH,
            )

        subrings = [
            (0, "R", 0, 0, N_R, left),
            (2, "L", 0, 0, N_L, right),
            (1, "R", 1, 1, N_R, left),
            (3, "L", 1, 1, N_L, right),
        ]
        pending = {}
        for j in range(N_R + 2):
            started = []
            for s, dirn, half, phase, n_dir, upstream in subrings:
                h = j - phase
                if h < 0 or h > n_dir:
                    continue
                if h >= 1:
                    pending[s].wait_recv()
                if h < n_dir:
                    if h >= 1:
                        pl.semaphore_wait(creds.at[s], 1)
                    rd = make_rdma(s, dirn, half, h)
                    rd.start()
                    pending[s] = rd
                    started.append((s, h, n_dir, upstream))
                k_base = half * E_HALF
                if h == 0:
                    if dirn == "R":
                        accum(my, ewb_ref[pl.ds(k_base, E_HALF)], k_base,
                              init=(half == 0))
                elif not _COMM_ONLY_PROBE:
                    if dirn == "R":
                        o = lut(ring_t, lax.rem(r - h + N_DEV, N_DEV))
                    else:
                        o = lut(ring_t, lax.rem(r + h, N_DEV))
                    accum(o, bufs[s, h % 2], k_base, init=False)
            for s, h, n_dir, upstream in started:
                pending[s].wait_send()
                if h <= n_dir - 2:
                    pl.semaphore_signal(
                        creds.at[s], inc=1,
                        device_id=(upstream,),
                        device_id_type=pl.DeviceIdType.MESH,
                    )

    return pl.pallas_call(
        body,
        out_shape=jax.ShapeDtypeStruct((T, H), jnp.float32),
        in_specs=[pl.BlockSpec(memory_space=pltpu.VMEM)] * 6,
        out_specs=pl.BlockSpec(memory_space=pltpu.VMEM),
        scratch_shapes=[
            pltpu.VMEM((E_LOC, D, H), jnp.bfloat16),
            pltpu.VMEM((4, 2, E_HALF, D, H), jnp.bfloat16),
            pltpu.SemaphoreType.DMA((4, 2)),
            pltpu.SemaphoreType.DMA((4, 2)),
            pltpu.SemaphoreType.REGULAR((4,)),
        ],
        compiler_params=pltpu.CompilerParams(collective_id=0),
    )(x, router_W, route_idx, expert_W,
      jnp.asarray(RING, dtype=jnp.int32).reshape(1, N_DEV),
      jnp.asarray(INV_RING, dtype=jnp.int32).reshape(1, N_DEV))


# device time: 117160 ns/iter; 1.0809x vs baseline; 1.0809x over previous
import sys

import jax
import jax.numpy as jnp
from jax import lax
from jax.experimental import pallas as pl
from jax.experimental.pallas import tpu as pltpu

N_DEV = 16
E_LOC = 4
E_HALF = 2

import os
_COMM_ONLY_PROBE = bool(os.environ.get("COMM_ONLY_PROBE"))


def _build_ring():
    try:
        import distributed_mesh_v7x as dm

        mesh = dm.get_mesh("i", N_DEV)
        coords = [tuple(d.coords) for d in mesh.devices.flat]
        xs = sorted({c[0] for c in coords})
        ys = sorted({c[1] for c in coords})
        zs = sorted({c[2] for c in coords})
        if (
            len(set(coords)) == N_DEV
            and len(xs) == 2
            and len(ys) == 2
            and len(zs) == 4
        ):
            path = [
                (0, 0, 0), (1, 0, 0), (1, 0, 1), (1, 0, 2),
                (1, 0, 3), (1, 1, 3), (1, 1, 2), (1, 1, 1),
                (1, 1, 0), (0, 1, 0), (0, 1, 1), (0, 1, 2),
                (0, 1, 3), (0, 0, 3), (0, 0, 2), (0, 0, 1),
            ]
            idx = {c: i for i, c in enumerate(coords)}
            ring = [idx[(xs[a], ys[b], zs[c])] for a, b, c in path]
            if sorted(ring) == list(range(N_DEV)):
                return ring
    except Exception as e:
        print(f"_build_ring fallback: {type(e).__name__}: {e}", file=sys.stderr)
    return list(range(N_DEV))


RING = _build_ring()
print(f"RING = {RING}", file=sys.stderr)
INV_RING = [0] * N_DEV
for _p, _l in enumerate(RING):
    INV_RING[_l] = _p


def kernel(x, router_W, route_idx, expert_W):
    T, D = x.shape
    E = router_W.shape[1]
    H = expert_W.shape[2]
    N_R = 8
    N_L = 7

    def body(x_ref, rw_ref, idx_ref, ew_ref, ring_ref, inv_ref, out_ref,
             ewb_ref, bufs, sends, recvs):
        my = lax.axis_index("i")
        i16 = lax.broadcasted_iota(jnp.int32, (1, N_DEV), 1)
        ring_t = ring_ref[:, :]
        inv_t = inv_ref[:, :]

        def lut(tbl, i):
            return jnp.sum(jnp.where(i16 == i, tbl, 0))

        r = lut(inv_t, my)
        right = lut(ring_t, lax.rem(r + 1, N_DEV))
        left = lut(ring_t, lax.rem(r + N_DEV - 1, N_DEV))

        barrier_sem = pltpu.get_barrier_semaphore()
        for nbr in (left, right):
            pl.semaphore_signal(
                barrier_sem, inc=1,
                device_id=(nbr,), device_id_type=pl.DeviceIdType.MESH,
            )
        pl.semaphore_wait(barrier_sem, 2)

        ewb_ref[:, :, :] = ew_ref[:, :, :].astype(jnp.bfloat16)

        xv = x_ref[:, :]
        xb = xv.astype(jnp.bfloat16)
        scores = jnp.dot(xv, rw_ref[:, :], preferred_element_type=jnp.float32)
        e0 = idx_ref[:, 0:1]
        e1 = idx_ref[:, 1:2]
        cols = lax.broadcasted_iota(jnp.int32, (T, E), 1)
        s0 = jnp.sum(jnp.where(cols == e0, scores, 0.0), axis=1, keepdims=True)
        s1 = jnp.sum(jnp.where(cols == e1, scores, 0.0), axis=1, keepdims=True)
        w0 = 1.0 / (1.0 + jnp.exp(s1 - s0))
        w1 = 1.0 - w0

        def accum(origin, w_half, k_base, init):
            contrib = None
            for k in range(E_HALF):
                e_id = origin * E_LOC + k_base + k
                col = (jnp.where(e0 == e_id, w0, 0.0)
                       + jnp.where(e1 == e_id, w1, 0.0))
                xs = xb * col.astype(jnp.bfloat16)
                part = jnp.dot(xs, w_half[k],
                               preferred_element_type=jnp.float32)
                contrib = part if contrib is None else contrib + part
            if init:
                out_ref[:, :] = contrib
            else:
                out_ref[:, :] += contrib

        def make_rdma(s, dirn, half, h):
            if h == 0:
                src = ewb_ref.at[pl.ds(half * E_HALF, E_HALF)]
            else:
                src = bufs.at[s, h - 1]
            return pltpu.make_async_remote_copy(
                src_ref=src,
                dst_ref=bufs.at[s, h],
                send_sem=sends.at[s, h % 2],
                recv_sem=recvs.at[s, h],
                device_id=(right if dirn == "R" else left,),
                device_id_type=pl.DeviceIdType.MESH,
            )

        subrings = [
            (0, "R", 0, 8),
            (2, "L", 0, 7),
            (1, "R", 1, 7),
            (3, "L", 1, 8),
        ]
        pending = {}
        for j in range(9):
            started = []
            for s, dirn, half, n_dir in subrings:
                h = j
                if h > n_dir:
                    continue
                if h >= 1:
                    pending[s].wait_recv()
                if h < n_dir:
                    rd = make_rdma(s, dirn, half, h)
                    rd.start()
                    pending[s] = rd
                    started.append(rd)
                k_base = half * E_HALF
                if h == 0:
                    if dirn == "R":
                        accum(my, ewb_ref[pl.ds(k_base, E_HALF)], k_base,
                              init=(half == 0))
                elif not _COMM_ONLY_PROBE:
                    if dirn == "R":
                        o = lut(ring_t, lax.rem(r - h + N_DEV, N_DEV))
                    else:
                        o = lut(ring_t, lax.rem(r + h, N_DEV))
                    accum(o, bufs[s, h - 1], k_base, init=False)
            for rd in started:
                rd.wait_send()

    return pl.pallas_call(
        body,
        out_shape=jax.ShapeDtypeStruct((T, H), jnp.float32),
        in_specs=[pl.BlockSpec(memory_space=pltpu.VMEM)] * 6,
        out_specs=pl.BlockSpec(memory_space=pltpu.VMEM),
        scratch_shapes=[
            pltpu.VMEM((E_LOC, D, H), jnp.bfloat16),
            pltpu.VMEM((4, 8, E_HALF, D, H), jnp.bfloat16),
            pltpu.SemaphoreType.DMA((4, 2)),
            pltpu.SemaphoreType.DMA((4, 8)),
        ],
        compiler_params=pltpu.CompilerParams(collective_id=0),
    )(x, router_W, route_idx, expert_W,
      jnp.asarray(RING, dtype=jnp.int32).reshape(1, N_DEV),
      jnp.asarray(INV_RING, dtype=jnp.int32).reshape(1, N_DEV))


# device time: 106581 ns/iter; 1.1882x vs baseline; 1.0993x over previous
import sys

import jax
import jax.numpy as jnp
from jax import lax
from jax.experimental import pallas as pl
from jax.experimental.pallas import tpu as pltpu

N_DEV = 16
E_LOC = 4
E_HALF = 2

import os
_COMM_ONLY_PROBE = bool(os.environ.get("COMM_ONLY_PROBE"))


def _build_ring():
    try:
        import distributed_mesh_v7x as dm

        mesh = dm.get_mesh("i", N_DEV)
        coords = [tuple(d.coords) for d in mesh.devices.flat]
        xs = sorted({c[0] for c in coords})
        ys = sorted({c[1] for c in coords})
        zs = sorted({c[2] for c in coords})
        if (
            len(set(coords)) == N_DEV
            and len(xs) == 2
            and len(ys) == 2
            and len(zs) == 4
        ):
            path = [
                (0, 0, 0), (1, 0, 0), (1, 0, 1), (1, 0, 2),
                (1, 0, 3), (1, 1, 3), (1, 1, 2), (1, 1, 1),
                (1, 1, 0), (0, 1, 0), (0, 1, 1), (0, 1, 2),
                (0, 1, 3), (0, 0, 3), (0, 0, 2), (0, 0, 1),
            ]
            idx = {c: i for i, c in enumerate(coords)}
            ring = [idx[(xs[a], ys[b], zs[c])] for a, b, c in path]
            if sorted(ring) == list(range(N_DEV)):
                return ring
    except Exception as e:
        print(f"_build_ring fallback: {type(e).__name__}: {e}", file=sys.stderr)
    return list(range(N_DEV))


RING = _build_ring()
print(f"RING = {RING}", file=sys.stderr)
INV_RING = [0] * N_DEV
for _p, _l in enumerate(RING):
    INV_RING[_l] = _p


def kernel(x, router_W, route_idx, expert_W):
    T, D = x.shape
    E = router_W.shape[1]
    H = expert_W.shape[2]
    N_R = 8
    N_L = 7

    def body(x_ref, rw_ref, idx_ref, ew_ref, ring_ref, inv_ref, out_ref,
             ewb_ref, bufs, sends, recvs):
        my = lax.axis_index("i")
        i16 = lax.broadcasted_iota(jnp.int32, (1, N_DEV), 1)
        ring_t = ring_ref[:, :]
        inv_t = inv_ref[:, :]

        def lut(tbl, i):
            return jnp.sum(jnp.where(i16 == i, tbl, 0))

        r = lut(inv_t, my)
        right = lut(ring_t, lax.rem(r + 1, N_DEV))
        left = lut(ring_t, lax.rem(r + N_DEV - 1, N_DEV))

        barrier_sem = pltpu.get_barrier_semaphore()
        for nbr in (left, right):
            pl.semaphore_signal(
                barrier_sem, inc=1,
                device_id=(nbr,), device_id_type=pl.DeviceIdType.MESH,
            )
        pl.semaphore_wait(barrier_sem, 2)

        ewb_ref[:, :, :] = ew_ref[:, :, :].astype(jnp.bfloat16)

        xv = x_ref[:, :]
        xb = xv.astype(jnp.bfloat16)
        scores = jnp.dot(xv, rw_ref[:, :], preferred_element_type=jnp.float32)
        e0 = idx_ref[:, 0:1]
        e1 = idx_ref[:, 1:2]
        cols = lax.broadcasted_iota(jnp.int32, (T, E), 1)
        s0 = jnp.sum(jnp.where(cols == e0, scores, 0.0), axis=1, keepdims=True)
        s1 = jnp.sum(jnp.where(cols == e1, scores, 0.0), axis=1, keepdims=True)
        w0 = 1.0 / (1.0 + jnp.exp(s1 - s0))
        w1 = 1.0 - w0

        def accum(origin, w_half, k_base, init):
            contrib = None
            for k in range(E_HALF):
                e_id = origin * E_LOC + k_base + k
                col = (jnp.where(e0 == e_id, w0, 0.0)
                       + jnp.where(e1 == e_id, w1, 0.0))
                xs = xb * col.astype(jnp.bfloat16)
                part = jnp.dot(xs, w_half[k],
                               preferred_element_type=jnp.float32)
                contrib = part if contrib is None else contrib + part
            if init:
                out_ref[:, :] = contrib
            else:
                out_ref[:, :] += contrib

        def make_rdma(s, dirn, half, h):
            if h == 0:
                src = ewb_ref.at[pl.ds(half * E_HALF, E_HALF)]
            else:
                src = bufs.at[s, h - 1]
            return pltpu.make_async_remote_copy(
                src_ref=src,
                dst_ref=bufs.at[s, h],
                send_sem=sends.at[s, h % 2],
                recv_sem=recvs.at[s, h],
                device_id=(right if dirn == "R" else left,),
                device_id_type=pl.DeviceIdType.MESH,
            )

        subrings = [
            (0, "R", 0, 8),
            (2, "L", 0, 7),
            (1, "R", 1, 7),
            (3, "L", 1, 8),
        ]
        pending = {}
        for j in range(9):
            started = []
            for s, dirn, half, n_dir in subrings:
                if j > n_dir:
                    continue
                if j >= 1:
                    pending[s].wait_recv()
                if j < n_dir:
                    rd = make_rdma(s, dirn, half, j)
                    rd.start()
                    pending[s] = rd
                    started.append(rd)
            for s, dirn, half, n_dir in subrings:
                if j > n_dir:
                    continue
                k_base = half * E_HALF
                if j == 0:
                    if dirn == "R":
                        accum(my, ewb_ref[pl.ds(k_base, E_HALF)], k_base,
                              init=(half == 0))
                elif not _COMM_ONLY_PROBE:
                    if dirn == "R":
                        o = lut(ring_t, lax.rem(r - j + N_DEV, N_DEV))
                    else:
                        o = lut(ring_t, lax.rem(r + j, N_DEV))
                    accum(o, bufs[s, j - 1], k_base, init=False)
            for rd in started:
                rd.wait_send()

    return pl.pallas_call(
        body,
        out_shape=jax.ShapeDtypeStruct((T, H), jnp.float32),
        in_specs=[pl.BlockSpec(memory_space=pltpu.VMEM)] * 6,
        out_specs=pl.BlockSpec(memory_space=pltpu.VMEM),
        scratch_shapes=[
            pltpu.VMEM((E_LOC, D, H), jnp.bfloat16),
            pltpu.VMEM((4, 8, E_HALF, D, H), jnp.bfloat16),
            pltpu.SemaphoreType.DMA((4, 2)),
            pltpu.SemaphoreType.DMA((4, 8)),
        ],
        compiler_params=pltpu.CompilerParams(collective_id=0),
    )(x, router_W, route_idx, expert_W,
      jnp.asarray(RING, dtype=jnp.int32).reshape(1, N_DEV),
      jnp.asarray(INV_RING, dtype=jnp.int32).reshape(1, N_DEV))


# device time: 105093 ns/iter; 1.2050x vs baseline; 1.0142x over previous
import sys

import jax
import jax.numpy as jnp
from jax import lax
from jax.experimental import pallas as pl
from jax.experimental.pallas import tpu as pltpu

N_DEV = 16
E_LOC = 4
E_HALF = 2

import os
_COMM_ONLY_PROBE = bool(os.environ.get("COMM_ONLY_PROBE"))


def _build_ring():
    try:
        import distributed_mesh_v7x as dm

        mesh = dm.get_mesh("i", N_DEV)
        coords = [tuple(d.coords) for d in mesh.devices.flat]
        xs = sorted({c[0] for c in coords})
        ys = sorted({c[1] for c in coords})
        zs = sorted({c[2] for c in coords})
        if (
            len(set(coords)) == N_DEV
            and len(xs) == 2
            and len(ys) == 2
            and len(zs) == 4
        ):
            path = [
                (0, 0, 0), (1, 0, 0), (1, 0, 1), (1, 0, 2),
                (1, 0, 3), (1, 1, 3), (1, 1, 2), (1, 1, 1),
                (1, 1, 0), (0, 1, 0), (0, 1, 1), (0, 1, 2),
                (0, 1, 3), (0, 0, 3), (0, 0, 2), (0, 0, 1),
            ]
            idx = {c: i for i, c in enumerate(coords)}
            ring = [idx[(xs[a], ys[b], zs[c])] for a, b, c in path]
            if sorted(ring) == list(range(N_DEV)):
                return ring
    except Exception as e:
        print(f"_build_ring fallback: {type(e).__name__}: {e}", file=sys.stderr)
    return list(range(N_DEV))


RING = _build_ring()
print(f"RING = {RING}", file=sys.stderr)
INV_RING = [0] * N_DEV
for _p, _l in enumerate(RING):
    INV_RING[_l] = _p


def kernel(x, router_W, route_idx, expert_W):
    T, D = x.shape
    E = router_W.shape[1]
    H = expert_W.shape[2]
    N_R = 8
    N_L = 7

    def body(x_ref, rw_ref, idx_ref, ew_ref, ring_ref, inv_ref, out_ref,
             ewb_ref, bufs, sends, recvs):
        my = lax.axis_index("i")
        i16 = lax.broadcasted_iota(jnp.int32, (1, N_DEV), 1)
        ring_t = ring_ref[:, :]
        inv_t = inv_ref[:, :]

        def lut(tbl, i):
            return jnp.sum(jnp.where(i16 == i, tbl, 0))

        r = lut(inv_t, my)
        right = lut(ring_t, lax.rem(r + 1, N_DEV))
        left = lut(ring_t, lax.rem(r + N_DEV - 1, N_DEV))

        ewb_ref[:, :, :] = ew_ref[:, :, :].astype(jnp.bfloat16)

        barrier_sem = pltpu.get_barrier_semaphore()
        for nbr in (left, right):
            pl.semaphore_signal(
                barrier_sem, inc=1,
                device_id=(nbr,), device_id_type=pl.DeviceIdType.MESH,
            )
        pl.semaphore_wait(barrier_sem, 2)

        _gates = []

        def gates():
            if not _gates:
                xv = x_ref[:, :]
                xb = xv.astype(jnp.bfloat16)
                scores = jnp.dot(xv, rw_ref[:, :],
                                 preferred_element_type=jnp.float32)
                e0 = idx_ref[:, 0:1]
                e1 = idx_ref[:, 1:2]
                cols = lax.broadcasted_iota(jnp.int32, (T, E), 1)
                s0 = jnp.sum(jnp.where(cols == e0, scores, 0.0),
                             axis=1, keepdims=True)
                s1 = jnp.sum(jnp.where(cols == e1, scores, 0.0),
                             axis=1, keepdims=True)
                w0 = 1.0 / (1.0 + jnp.exp(s1 - s0))
                w1 = 1.0 - w0
                _gates.append((xb, e0, e1, w0, w1))
            return _gates[0]

        def accum(origin, w_half, k_base, init):
            xb, e0, e1, w0, w1 = gates()
            contrib = None
            for k in range(E_HALF):
                e_id = origin * E_LOC + k_base + k
                col = (jnp.where(e0 == e_id, w0, 0.0)
                       + jnp.where(e1 == e_id, w1, 0.0))
                xs = xb * col.astype(jnp.bfloat16)
                part = jnp.dot(xs, w_half[k],
                               preferred_element_type=jnp.float32)
                contrib = part if contrib is None else contrib + part
            if init:
                out_ref[:, :] = contrib
            else:
                out_ref[:, :] += contrib

        def make_rdma(s, dirn, half, h):
            if h == 0:
                src = ewb_ref.at[pl.ds(half * E_HALF, E_HALF)]
            else:
                src = bufs.at[s, h - 1]
            return pltpu.make_async_remote_copy(
                src_ref=src,
                dst_ref=bufs.at[s, h],
                send_sem=sends.at[s, h % 2],
                recv_sem=recvs.at[s, h],
                device_id=(right if dirn == "R" else left,),
                device_id_type=pl.DeviceIdType.MESH,
            )

        subrings = [
            (0, "R", 0, 8),
            (2, "L", 0, 7),
            (1, "R", 1, 7),
            (3, "L", 1, 8),
        ]
        pending = {}
        for j in range(9):
            started = []
            for s, dirn, half, n_dir in subrings:
                if j > n_dir:
                    continue
                if j >= 1:
                    pending[s].wait_recv()
                if j < n_dir:
                    rd = make_rdma(s, dirn, half, j)
                    rd.start()
                    pending[s] = rd
                    started.append(rd)
            for s, dirn, half, n_dir in subrings:
                if j > n_dir:
                    continue
                k_base = half * E_HALF
                if j == 0:
                    if dirn == "R":
                        accum(my, ewb_ref[pl.ds(k_base, E_HALF)], k_base,
                              init=(half == 0))
                elif not _COMM_ONLY_PROBE:
                    if dirn == "R":
                        o = lut(ring_t, lax.rem(r - j + N_DEV, N_DEV))
                    else:
                        o = lut(ring_t, lax.rem(r + j, N_DEV))
                    accum(o, bufs[s, j - 1], k_base, init=False)
            for rd in started:
                rd.wait_send()

    return pl.pallas_call(
        body,
        out_shape=jax.ShapeDtypeStruct((T, H), jnp.float32),
        in_specs=[pl.BlockSpec(memory_space=pltpu.VMEM)] * 6,
        out_specs=pl.BlockSpec(memory_space=pltpu.VMEM),
        scratch_shapes=[
            pltpu.VMEM((E_LOC, D, H), jnp.bfloat16),
            pltpu.VMEM((4, 8, E_HALF, D, H), jnp.bfloat16),
            pltpu.SemaphoreType.DMA((4, 2)),
            pltpu.SemaphoreType.DMA((4, 8)),
        ],
        compiler_params=pltpu.CompilerParams(collective_id=0),
    )(x, router_W, route_idx, expert_W,
      jnp.asarray(RING, dtype=jnp.int32).reshape(1, N_DEV),
      jnp.asarray(INV_RING, dtype=jnp.int32).reshape(1, N_DEV))
